# baseline (speedup 1.0000x reference)
import jax
import jax.numpy as jnp
from jax import lax
from jax.experimental import pallas as pl


_NUM_CLASSES = 4
_IN_BLK = 128
_OUT_BLK = _IN_BLK * _NUM_CLASSES


def _onehot_body(x_ref, o_ref):
    r = x_ref.shape[0]
    jq = lax.broadcasted_iota(jnp.int32, (_IN_BLK, _OUT_BLK), 1)
    jj = lax.broadcasted_iota(jnp.int32, (_IN_BLK, _OUT_BLK), 0)
    e = ((jq // _NUM_CLASSES) == jj).astype(jnp.bfloat16)
    xb = x_ref[...].astype(jnp.bfloat16)
    xrep = jnp.dot(xb, e, preferred_element_type=jnp.float32)
    cpat = (
        lax.broadcasted_iota(jnp.int32, (r, _OUT_BLK), 1) % _NUM_CLASSES
    ).astype(jnp.float32)
    o_ref[...] = (xrep == cpat).astype(jnp.float32)


def kernel(x):
    n, m = x.shape
    out2d = pl.pallas_call(
        _onehot_body,
        grid=(m // _IN_BLK,),
        in_specs=[pl.BlockSpec((n, _IN_BLK), lambda t: (0, t))],
        out_specs=pl.BlockSpec((n, _OUT_BLK), lambda t: (0, t)),
        out_shape=jax.ShapeDtypeStruct((n, m * _NUM_CLASSES), jnp.float32),
    )(x)
    return out2d.reshape(n, m, _NUM_CLASSES)


# speedup vs baseline: 3.0232x; 3.0232x over previous
import jax
import jax.numpy as jnp
from jax.experimental import pallas as pl


_NUM_CLASSES = 4
_LANES = 128
_ROW_BLK = 128


def _onehot_body(x_ref, o_ref):
    m = x_ref.shape[1]
    xv = x_ref[...]
    for jt in range(m // _LANES):
        xs = xv[:, jt * _LANES:(jt + 1) * _LANES]
        for c in range(_NUM_CLASSES):
            o_ref[:, _NUM_CLASSES * jt + c, :] = (xs == c).astype(jnp.float32)


def kernel(x):
    n, m = x.shape
    jt = m // _LANES
    o = pl.pallas_call(
        _onehot_body,
        grid=(n // _ROW_BLK,),
        in_specs=[pl.BlockSpec((_ROW_BLK, m), lambda i: (i, 0))],
        out_specs=pl.BlockSpec(
            (_ROW_BLK, jt * _NUM_CLASSES, _LANES), lambda i: (i, 0, 0)
        ),
        out_shape=jax.ShapeDtypeStruct((n, jt * _NUM_CLASSES, _LANES), jnp.float32),
    )(x)
    return (
        o.reshape(n, jt, _NUM_CLASSES, _LANES)
        .transpose(0, 1, 3, 2)
        .reshape(n, m, _NUM_CLASSES)
    )


# speedup vs baseline: 7.5550x; 2.4990x over previous
import jax
from jax import lax
import jax.numpy as jnp
from jax.experimental import pallas as pl


_NUM_CLASSES = 4
_LANES = 128
_ROW_BLK = 128


def _onehot_body(x_ref, o_ref):
    r, m = x_ref.shape
    jt = m // _LANES
    xv = x_ref[...].reshape(r, jt, _LANES)
    xrep = jnp.repeat(xv, _NUM_CLASSES, axis=1)
    ci = lax.broadcasted_iota(jnp.int32, (r, jt * _NUM_CLASSES, _LANES), 1)
    o_ref[...] = (xrep == ci % _NUM_CLASSES).astype(jnp.float32)


def kernel(x):
    n, m = x.shape
    jt = m // _LANES
    o = pl.pallas_call(
        _onehot_body,
        grid=(n // _ROW_BLK,),
        in_specs=[pl.BlockSpec((_ROW_BLK, m), lambda i: (i, 0))],
        out_specs=pl.BlockSpec(
            (_ROW_BLK, jt * _NUM_CLASSES, _LANES), lambda i: (i, 0, 0)
        ),
        out_shape=jax.ShapeDtypeStruct((n, jt * _NUM_CLASSES, _LANES), jnp.float32),
    )(x)
    return (
        o.reshape(n, jt, _NUM_CLASSES, _LANES)
        .transpose(0, 1, 3, 2)
        .reshape(n, m, _NUM_CLASSES)
    )
